# Initial kernel scaffold; baseline (speedup 1.0000x reference)
import jax
import jax.numpy as jnp
from jax.experimental import pallas as pl


def kernel(ctx, cnt_table, val_table, W, b):
    raise NotImplementedError("write your pallas kernel here")


# speedup vs baseline: 3.9483x; 3.9483x over previous
import functools

import jax
import jax.numpy as jnp
from jax import lax
from jax.experimental import pallas as pl
from jax.experimental.pallas import tpu as pltpu
from jax.experimental.pallas import tpu_sc as plsc

K = 26
NEMBED = 64
BATCH = 16384
NHID = 128
KN = K * NEMBED

NC, NS, L = 2, 16, 16
NW = NC * NS
BW = BATCH // NW
CHUNK = 128
NCH = BW // CHUNK


def _sc_build():
    mesh = plsc.VectorSubcoreMesh(core_axis_name="c", subcore_axis_name="s")

    @functools.partial(
        pl.kernel,
        mesh=mesh,
        compiler_params=pltpu.CompilerParams(use_tc_tiling_on_sc=False),
        out_type=jax.ShapeDtypeStruct((BATCH, KN), jnp.float32),
        scratch_types=[
            pltpu.VMEM((CHUNK,), jnp.int32),
            pltpu.VMEM((CHUNK,), jnp.int32),
            pltpu.VMEM((CHUNK, NEMBED), jnp.float32),
            pltpu.VMEM((CHUNK, NEMBED), jnp.float32),
            pltpu.SemaphoreType.DMA,
            pltpu.SemaphoreType.DMA,
        ],
    )
    def sc_gather_mul(ctx_r, cnt_t, val_t, h_out, idx_c, idx_v, rows_c, rows_v,
                      sem_c, sem_v):
        wid = lax.axis_index("s") * NC + lax.axis_index("c")

        @pl.loop(0, K)
        def _k(k):
            @pl.loop(0, NCH)
            def _j(j):
                row = wid * NCH + j
                b0 = row * CHUNK
                pltpu.sync_copy(ctx_r.at[2 * k, row], idx_c)
                pltpu.sync_copy(ctx_r.at[2 * k + 1, row], idx_v)
                cp1 = pltpu.async_copy(cnt_t.at[idx_c], rows_c, sem_c)
                cp2 = pltpu.async_copy(val_t.at[idx_v], rows_v, sem_v)
                cp1.wait()
                cp2.wait()

                @pl.loop(0, CHUNK, unroll=8)
                def _m(r):
                    for m in range(NEMBED // L):
                        s = pl.ds(m * L, L)
                        rows_c[r, s] = rows_c[r, s] * rows_v[r, s]

                pltpu.sync_copy(
                    rows_c,
                    h_out.at[pl.ds(b0, CHUNK), pl.ds(k * NEMBED, NEMBED)],
                )

    return sc_gather_mul


_sc_gather_mul = _sc_build()


def _tc_body(h_ref, w_ref, b_ref, o_ref):
    o_ref[:] = jnp.tanh(h_ref[:]) @ w_ref[:] + b_ref[:]


def _tc_mlp(h, W, b2):
    bB = 1024
    return pl.pallas_call(
        _tc_body,
        grid=(BATCH // bB,),
        in_specs=[
            pl.BlockSpec((bB, KN), lambda i: (i, 0)),
            pl.BlockSpec((KN, NHID), lambda i: (0, 0)),
            pl.BlockSpec((1, NHID), lambda i: (0, 0)),
        ],
        out_specs=pl.BlockSpec((bB, NHID), lambda i: (i, 0)),
        out_shape=jax.ShapeDtypeStruct((BATCH, NHID), jnp.float32),
    )(h, W, b2)


def kernel(ctx, cnt_table, val_table, W, b):
    ctx_r = ctx.reshape(2 * K, BATCH // CHUNK, CHUNK)
    h = _sc_gather_mul(ctx_r, cnt_table, val_table)
    out = _tc_mlp(h, W, b.reshape(1, NHID))
    return out[None]


# speedup vs baseline: 5.4831x; 1.3887x over previous
import functools

import jax
import jax.numpy as jnp
from jax import lax
from jax.experimental import pallas as pl
from jax.experimental.pallas import tpu as pltpu
from jax.experimental.pallas import tpu_sc as plsc

K = 26
NEMBED = 64
BATCH = 16384
NHID = 128
KN = K * NEMBED

NC, NS, L = 2, 16, 16
NW = NC * NS
BW = BATCH // NW
CHUNK = 128
NCH = BW // CHUNK
T = K * NCH


def _sc_build():
    mesh = plsc.VectorSubcoreMesh(core_axis_name="c", subcore_axis_name="s", num_cores=NC, num_subcores=NS)

    @functools.partial(
        pl.kernel,
        mesh=mesh,
        compiler_params=pltpu.CompilerParams(use_tc_tiling_on_sc=False),
        out_type=jax.ShapeDtypeStruct((BATCH, KN), jnp.float32),
        scratch_types=[
            pltpu.VMEM((2 * K, NCH, CHUNK), jnp.int32),
            pltpu.VMEM((CHUNK, NEMBED), jnp.float32),
            pltpu.VMEM((CHUNK, NEMBED), jnp.float32),
            pltpu.VMEM((CHUNK, NEMBED), jnp.float32),
            pltpu.VMEM((CHUNK, NEMBED), jnp.float32),
            pltpu.SemaphoreType.DMA,
            pltpu.SemaphoreType.DMA,
            pltpu.SemaphoreType.DMA,
            pltpu.SemaphoreType.DMA,
        ],
    )
    def sc_gather_mul(ctx_r, cnt_t, val_t, h_out, idx_all,
                      rc0, rv0, rc1, rv1, sg0, sg1, sw0, sw1):
        wid = lax.axis_index("s") * NC + lax.axis_index("c")

        pltpu.sync_copy(ctx_r.at[:, pl.ds(wid * NCH, NCH), :], idx_all)

        bufs = ((rc0, rv0, sg0, sw0), (rc1, rv1, sg1, sw1))

        def issue_gathers(t, rc, rv, sg):
            k = lax.shift_right_logical(t, 2)
            j = lax.bitwise_and(t, 3)
            pltpu.async_copy(cnt_t.at[idx_all.at[2 * k, j]], rc, sg)
            pltpu.async_copy(val_t.at[idx_all.at[2 * k + 1, j]], rv, sg)

        def wait_gathers(rc, rv, sg):
            pltpu.make_async_copy(cnt_t.at[idx_all.at[0, 0]], rc, sg).wait()
            pltpu.make_async_copy(val_t.at[idx_all.at[0, 0]], rv, sg).wait()

        def h_slice(t):
            k = lax.shift_right_logical(t, 2)
            j = lax.bitwise_and(t, 3)
            b0 = (wid * NCH + j) * CHUNK
            return h_out.at[pl.ds(b0, CHUNK), pl.ds(k * NEMBED, NEMBED)]

        def wait_write(t, rc, sw):
            pltpu.make_async_copy(rc, h_slice(t), sw).wait()

        issue_gathers(0, rc0, rv0, sg0)

        @pl.loop(0, T, step=2)
        def _t0(t0):
            for b in range(2):
                rc, rv, sg, sw = bufs[b]
                orc, orv, osg, osw = bufs[1 - b]
                t = t0 + b

                @pl.when(t >= 1)
                def _():
                    wait_write(t - 1, orc, osw)

                @pl.when(t + 1 < T)
                def _():
                    issue_gathers(t + 1, orc, orv, osg)

                wait_gathers(rc, rv, sg)

                @pl.loop(0, CHUNK, unroll=8)
                def _m(r):
                    for m in range(NEMBED // L):
                        s = pl.ds(m * L, L)
                        rc[r, s] = rc[r, s] * rv[r, s]

                pltpu.async_copy(rc, h_slice(t), sw)

        wait_write(T - 1, bufs[(T - 1) % 2][0], bufs[(T - 1) % 2][3])

    return sc_gather_mul


_sc_gather_mul = _sc_build()


def _tc_body(h_ref, w_ref, b_ref, o_ref):
    o_ref[:] = jnp.tanh(h_ref[:]) @ w_ref[:] + b_ref[:]


def _tc_mlp(h, W, b2):
    bB = 1024
    return pl.pallas_call(
        _tc_body,
        grid=(BATCH // bB,),
        in_specs=[
            pl.BlockSpec((bB, KN), lambda i: (i, 0)),
            pl.BlockSpec((KN, NHID), lambda i: (0, 0)),
            pl.BlockSpec((1, NHID), lambda i: (0, 0)),
        ],
        out_specs=pl.BlockSpec((bB, NHID), lambda i: (i, 0)),
        out_shape=jax.ShapeDtypeStruct((BATCH, NHID), jnp.float32),
    )(h, W, b2)


def kernel(ctx, cnt_table, val_table, W, b):
    ctx_r = ctx.reshape(2 * K, BATCH // CHUNK, CHUNK)
    h = _sc_gather_mul(ctx_r, cnt_table, val_table)
    out = _tc_mlp(h, W, b.reshape(1, NHID))
    return out[None]


# speedup vs baseline: 5.8576x; 1.0683x over previous
import functools

import jax
import jax.numpy as jnp
from jax import lax
from jax.experimental import pallas as pl
from jax.experimental.pallas import tpu as pltpu
from jax.experimental.pallas import tpu_sc as plsc

K = 26
NEMBED = 64
BATCH = 16384
NHID = 128
KN = K * NEMBED

NC, NS, L = 2, 16, 16
NW = NC * NS
CHUNK = 128
NSLAB = 2
SLAB = BATCH // NSLAB
BW = SLAB // NW
NCH = BW // CHUNK
T = K * NCH


def _sc_build(slab_start):
    mesh = plsc.VectorSubcoreMesh(
        core_axis_name="c", subcore_axis_name="s",
        num_cores=NC, num_subcores=NS)

    @functools.partial(
        pl.kernel,
        mesh=mesh,
        compiler_params=pltpu.CompilerParams(use_tc_tiling_on_sc=False),
        out_type=jax.ShapeDtypeStruct((SLAB, KN), jnp.float32),
        scratch_types=[
            pltpu.VMEM((2 * K, NCH, CHUNK), jnp.int32),
            pltpu.VMEM((CHUNK, NEMBED), jnp.float32),
            pltpu.VMEM((CHUNK, NEMBED), jnp.float32),
            pltpu.VMEM((CHUNK, NEMBED), jnp.float32),
            pltpu.VMEM((CHUNK, NEMBED), jnp.float32),
            pltpu.SemaphoreType.DMA,
            pltpu.SemaphoreType.DMA,
            pltpu.SemaphoreType.DMA,
            pltpu.SemaphoreType.DMA,
        ],
    )
    def sc_gather_mul(ctx_r, cnt_t, val_t, h_out, idx_all,
                      rc0, rv0, rc1, rv1, sg0, sg1, sw0, sw1):
        wid = lax.axis_index("s") * NC + lax.axis_index("c")
        crow = slab_start // CHUNK + wid * NCH

        pltpu.sync_copy(ctx_r.at[:, pl.ds(crow, NCH), :], idx_all)

        bufs = ((rc0, rv0, sg0, sw0), (rc1, rv1, sg1, sw1))

        def kj(t):
            if NCH == 2:
                return lax.shift_right_logical(t, 1), lax.bitwise_and(t, 1)
            return lax.shift_right_logical(t, 2), lax.bitwise_and(t, 3)

        def issue_gathers(t, rc, rv, sg):
            k, j = kj(t)
            pltpu.async_copy(cnt_t.at[idx_all.at[2 * k, j]], rc, sg)
            pltpu.async_copy(val_t.at[idx_all.at[2 * k + 1, j]], rv, sg)

        def wait_gathers(rc, rv, sg):
            pltpu.make_async_copy(cnt_t.at[idx_all.at[0, 0]], rc, sg).wait()
            pltpu.make_async_copy(val_t.at[idx_all.at[0, 0]], rv, sg).wait()

        def h_slice(t):
            k, j = kj(t)
            b0 = (wid * NCH + j) * CHUNK
            return h_out.at[pl.ds(b0, CHUNK), pl.ds(k * NEMBED, NEMBED)]

        def wait_write(t, rc, sw):
            pltpu.make_async_copy(rc, h_slice(t), sw).wait()

        issue_gathers(0, rc0, rv0, sg0)

        @pl.loop(0, T, step=2)
        def _t0(t0):
            for b in range(2):
                rc, rv, sg, sw = bufs[b]
                orc, orv, osg, osw = bufs[1 - b]
                t = t0 + b

                @pl.when(t >= 1)
                def _():
                    wait_write(t - 1, orc, osw)

                @pl.when(t + 1 < T)
                def _():
                    issue_gathers(t + 1, orc, orv, osg)

                wait_gathers(rc, rv, sg)

                @pl.loop(0, CHUNK, unroll=8)
                def _m(r):
                    for m in range(NEMBED // L):
                        s = pl.ds(m * L, L)
                        rc[r, s] = rc[r, s] * rv[r, s]

                pltpu.async_copy(rc, h_slice(t), sw)

        wait_write(T - 1, bufs[(T - 1) % 2][0], bufs[(T - 1) % 2][3])

    return sc_gather_mul


_sc_slabs = tuple(_sc_build(s * SLAB) for s in range(NSLAB))


def _tc_body(h_ref, w_ref, b_ref, o_ref):
    th = jnp.tanh(h_ref[:]).astype(jnp.bfloat16)
    acc = jax.lax.dot_general(
        th, w_ref[:], (((1,), (0,)), ((), ())),
        preferred_element_type=jnp.float32)
    o_ref[:] = acc + b_ref[:]


def _tc_mlp(h, W16, b2):
    bB = 1024
    return pl.pallas_call(
        _tc_body,
        grid=(SLAB // bB,),
        in_specs=[
            pl.BlockSpec((bB, KN), lambda i: (i, 0)),
            pl.BlockSpec((KN, NHID), lambda i: (0, 0)),
            pl.BlockSpec((1, NHID), lambda i: (0, 0)),
        ],
        out_specs=pl.BlockSpec((bB, NHID), lambda i: (i, 0)),
        out_shape=jax.ShapeDtypeStruct((SLAB, NHID), jnp.float32),
    )(h, W16, b2)


def kernel(ctx, cnt_table, val_table, W, b):
    ctx_r = ctx.reshape(2 * K, BATCH // CHUNK, CHUNK)
    W16 = W.astype(jnp.bfloat16)
    b2 = b.reshape(1, NHID)
    outs = []
    for s in range(NSLAB):
        h = _sc_slabs[s](ctx_r, cnt_table, val_table)
        outs.append(_tc_mlp(h, W16, b2))
    return jnp.concatenate(outs, axis=0)[None]


# speedup vs baseline: 5.8581x; 1.0001x over previous
import functools

import jax
import jax.numpy as jnp
from jax import lax
from jax.experimental import pallas as pl
from jax.experimental.pallas import tpu as pltpu
from jax.experimental.pallas import tpu_sc as plsc

K = 26
NEMBED = 64
BATCH = 16384
NHID = 128
KN = K * NEMBED

NC, NS, L = 2, 16, 16
NW = NC * NS
CHUNK = 128
NSLAB = 2
SLAB = BATCH // NSLAB
BW = SLAB // NW
NCH = BW // CHUNK
T = K * NCH


def _sc_build(slab_start):
    mesh = plsc.VectorSubcoreMesh(
        core_axis_name="c", subcore_axis_name="s",
        num_cores=NC, num_subcores=NS)

    @functools.partial(
        pl.kernel,
        mesh=mesh,
        compiler_params=pltpu.CompilerParams(use_tc_tiling_on_sc=False),
        out_type=jax.ShapeDtypeStruct((SLAB, KN), jnp.float32),
        scratch_types=[
            pltpu.VMEM((2 * K, NCH, CHUNK), jnp.int32),
            pltpu.VMEM((CHUNK, NEMBED), jnp.float32),
            pltpu.VMEM((CHUNK, NEMBED), jnp.float32),
            pltpu.VMEM((CHUNK, NEMBED), jnp.float32),
            pltpu.VMEM((CHUNK, NEMBED), jnp.float32),
            pltpu.SemaphoreType.DMA,
            pltpu.SemaphoreType.DMA,
            pltpu.SemaphoreType.DMA,
            pltpu.SemaphoreType.DMA,
        ],
    )
    def sc_gather_mul(ctx_r, cnt_t, val_t, h_out, idx_all,
                      rc0, rv0, rc1, rv1, sg0, sg1, sw0, sw1):
        wid = lax.axis_index("s") * NC + lax.axis_index("c")
        crow = slab_start // CHUNK + wid * NCH

        pltpu.sync_copy(ctx_r.at[:, pl.ds(crow, NCH), :], idx_all)

        bufs = ((rc0, rv0, sg0, sw0), (rc1, rv1, sg1, sw1))

        def kj(t):
            if NCH == 2:
                return lax.shift_right_logical(t, 1), lax.bitwise_and(t, 1)
            return lax.shift_right_logical(t, 2), lax.bitwise_and(t, 3)

        def issue_gathers(t, rc, rv, sg):
            k, j = kj(t)
            pltpu.async_copy(cnt_t.at[idx_all.at[2 * k, j]], rc, sg)
            pltpu.async_copy(val_t.at[idx_all.at[2 * k + 1, j]], rv, sg)

        def wait_gathers(rc, rv, sg):
            pltpu.make_async_copy(cnt_t.at[idx_all.at[0, 0]], rc, sg).wait()
            pltpu.make_async_copy(val_t.at[idx_all.at[0, 0]], rv, sg).wait()

        def h_slice(t):
            k, j = kj(t)
            b0 = (wid * NCH + j) * CHUNK
            return h_out.at[pl.ds(b0, CHUNK), pl.ds(k * NEMBED, NEMBED)]

        def wait_write(t, rc, sw):
            pltpu.make_async_copy(rc, h_slice(t), sw).wait()

        issue_gathers(0, rc0, rv0, sg0)

        @pl.loop(0, T, step=2)
        def _t0(t0):
            for b in range(2):
                rc, rv, sg, sw = bufs[b]
                orc, orv, osg, osw = bufs[1 - b]
                t = t0 + b

                @pl.when(t >= 1)
                def _():
                    wait_write(t - 1, orc, osw)

                @pl.when(t + 1 < T)
                def _():
                    issue_gathers(t + 1, orc, orv, osg)

                wait_gathers(rc, rv, sg)

                @pl.loop(0, CHUNK, unroll=8)
                def _m(r):
                    for m in range(NEMBED // L):
                        s = pl.ds(m * L, L)
                        rc[r, s] = rc[r, s] * rv[r, s]

                pltpu.async_copy(rc, h_slice(t), sw)

        wait_write(T - 1, bufs[(T - 1) % 2][0], bufs[(T - 1) % 2][3])

    return sc_gather_mul


_sc_slabs = tuple(_sc_build(s * SLAB) for s in range(NSLAB))


def _tc_body(h_ref, w_ref, b_ref, o_ref):
    th = jnp.tanh(h_ref[:]).astype(jnp.bfloat16)
    acc = jax.lax.dot_general(
        th, w_ref[:], (((1,), (0,)), ((), ())),
        preferred_element_type=jnp.float32)
    o_ref[:] = acc + b_ref[:]


def _tc_mlp(h, W16, b2):
    bB = 1024
    return pl.pallas_call(
        _tc_body,
        grid=(SLAB // bB,),
        in_specs=[
            pl.BlockSpec((bB, KN), lambda i: (i, 0)),
            pl.BlockSpec((KN, NHID), lambda i: (0, 0)),
            pl.BlockSpec((1, NHID), lambda i: (0, 0)),
        ],
        out_specs=pl.BlockSpec((bB, NHID), lambda i: (i, 0)),
        out_shape=jax.ShapeDtypeStruct((SLAB, NHID), jnp.float32),
    )(h, W16, b2)


def kernel(ctx, cnt_table, val_table, W, b):
    ctx_r = ctx.reshape(2 * K, BATCH // CHUNK, CHUNK)
    W16 = W.astype(jnp.bfloat16)
    b2 = b.reshape(1, NHID)
    hs = [_sc_slabs[s](ctx_r, cnt_table, val_table) for s in range(NSLAB)]
    outs = [_tc_mlp(h, W16, b2) for h in hs]
    return jnp.concatenate(outs, axis=0)[None]


# speedup vs baseline: 6.8031x; 1.1613x over previous
import functools

import jax
import jax.numpy as jnp
from jax import lax
from jax.experimental import pallas as pl
from jax.experimental.pallas import tpu as pltpu
from jax.experimental.pallas import tpu_sc as plsc

K = 26
NEMBED = 64
BATCH = 16384
NHID = 128
KN = K * NEMBED
NCG = KN // 128

NC, NS, L = 2, 16, 16
NW = NC * NS
CHUNK = 128
NSLAB = 2
SLAB = BATCH // NSLAB
BW = SLAB // NW
NCH = BW // CHUNK
T = K * NCH


def _sc_build(slab_start):
    mesh = plsc.VectorSubcoreMesh(
        core_axis_name="c", subcore_axis_name="s",
        num_cores=NC, num_subcores=NS)

    @functools.partial(
        pl.kernel,
        mesh=mesh,
        compiler_params=pltpu.CompilerParams(use_tc_tiling_on_sc=False),
        out_type=jax.ShapeDtypeStruct((NCG, SLAB, 128), jnp.float32),
        scratch_types=[
            pltpu.VMEM((2 * K, NCH, CHUNK), jnp.int32),
            pltpu.VMEM((CHUNK, NEMBED), jnp.float32),
            pltpu.VMEM((CHUNK, NEMBED), jnp.float32),
            pltpu.VMEM((CHUNK, NEMBED), jnp.float32),
            pltpu.VMEM((CHUNK, NEMBED), jnp.float32),
            pltpu.SemaphoreType.DMA,
            pltpu.SemaphoreType.DMA,
            pltpu.SemaphoreType.DMA,
            pltpu.SemaphoreType.DMA,
        ],
    )
    def sc_gather_mul(ctx_r, cnt_t, val_t, h_out, idx_all,
                      rc0, rv0, rc1, rv1, sg0, sg1, sw0, sw1):
        wid = lax.axis_index("s") * NC + lax.axis_index("c")
        crow = slab_start // CHUNK + wid * NCH

        pltpu.sync_copy(ctx_r.at[:, pl.ds(crow, NCH), :], idx_all)

        bufs = ((rc0, rv0, sg0, sw0), (rc1, rv1, sg1, sw1))

        def kj(t):
            return lax.shift_right_logical(t, 1), lax.bitwise_and(t, 1)

        def issue_gathers(t, rc, rv, sg):
            k, j = kj(t)
            pltpu.async_copy(cnt_t.at[idx_all.at[2 * k, j]], rc, sg)
            pltpu.async_copy(val_t.at[idx_all.at[2 * k + 1, j]], rv, sg)

        def wait_gathers(rc, rv, sg):
            pltpu.make_async_copy(cnt_t.at[idx_all.at[0, 0]], rc, sg).wait()
            pltpu.make_async_copy(val_t.at[idx_all.at[0, 0]], rv, sg).wait()

        def h_slice(t):
            k, j = kj(t)
            cg = lax.shift_right_logical(k, 1)
            half = lax.bitwise_and(k, 1) * NEMBED
            b0 = (wid * NCH + j) * CHUNK
            return h_out.at[cg, pl.ds(b0, CHUNK), pl.ds(half, NEMBED)]

        def wait_write(t, rc, sw):
            pltpu.make_async_copy(rc, h_slice(t), sw).wait()

        issue_gathers(0, rc0, rv0, sg0)

        @pl.loop(0, T, step=2)
        def _t0(t0):
            for b in range(2):
                rc, rv, sg, sw = bufs[b]
                orc, orv, osg, osw = bufs[1 - b]
                t = t0 + b

                @pl.when(t >= 1)
                def _():
                    wait_write(t - 1, orc, osw)

                @pl.when(t + 1 < T)
                def _():
                    issue_gathers(t + 1, orc, orv, osg)

                wait_gathers(rc, rv, sg)

                @pl.loop(0, CHUNK, unroll=8)
                def _m(r):
                    for m in range(NEMBED // L):
                        s = pl.ds(m * L, L)
                        rc[r, s] = rc[r, s] * rv[r, s]

                pltpu.async_copy(rc, h_slice(t), sw)

        wait_write(T - 1, bufs[(T - 1) % 2][0], bufs[(T - 1) % 2][3])

    return sc_gather_mul


_sc_slabs = tuple(_sc_build(s * SLAB) for s in range(NSLAB))


def _tc_body(h_ref, w_ref, b_ref, o_ref):
    acc = jnp.zeros(o_ref.shape, jnp.float32)
    for c in range(NCG):
        th = jnp.tanh(h_ref[c]).astype(jnp.bfloat16)
        acc += jax.lax.dot_general(
            th, w_ref[c], (((1,), (0,)), ((), ())),
            preferred_element_type=jnp.float32)
    o_ref[:] = acc + b_ref[:]


def _tc_mlp(h3, W3, b2):
    bB = 1024
    return pl.pallas_call(
        _tc_body,
        grid=(SLAB // bB,),
        in_specs=[
            pl.BlockSpec((NCG, bB, 128), lambda i: (0, i, 0)),
            pl.BlockSpec((NCG, 128, NHID), lambda i: (0, 0, 0)),
            pl.BlockSpec((1, NHID), lambda i: (0, 0)),
        ],
        out_specs=pl.BlockSpec((bB, NHID), lambda i: (i, 0)),
        out_shape=jax.ShapeDtypeStruct((SLAB, NHID), jnp.float32),
    )(h3, W3, b2)


def kernel(ctx, cnt_table, val_table, W, b):
    ctx_r = ctx.reshape(2 * K, BATCH // CHUNK, CHUNK)
    W3 = W.reshape(NCG, 128, NHID).astype(jnp.bfloat16)
    b2 = b.reshape(1, NHID)
    hs = [_sc_slabs[s](ctx_r, cnt_table, val_table) for s in range(NSLAB)]
    outs = [_tc_mlp(h3, W3, b2) for h3 in hs]
    return jnp.concatenate(outs, axis=0)[None]
